# Initial kernel scaffold; baseline (speedup 1.0000x reference)
import jax
import jax.numpy as jnp
from jax.experimental import pallas as pl


def kernel(x, edge_index, Wl1, bl1, Wr1, g1, be1, m1, v1, Wl2, bl2, Wr2, g2, be2, m2, v2):
    raise NotImplementedError("write your pallas kernel here")


# speedup vs baseline: 6.3950x; 6.3950x over previous
import functools

import jax
import jax.numpy as jnp
from jax import lax
from jax.experimental import pallas as pl
from jax.experimental.pallas import tpu as pltpu
from jax.experimental.pallas import tpu_sc as plsc

_NC = 2
_NS = 16
_C = 128


def _sc_scatter(x, src, dst):
    n, d = x.shape
    e = src.shape[0]
    assert e % (_NC * _C) == 0
    chunks_per_core = e // (_NC * _C)
    tmax = (chunks_per_core + _NS - 1) // _NS
    rpt = (n // _NS) // 8 * 8
    rem = n - rpt * _NS
    assert rem % 8 == 0 and rem <= rpt
    mesh = plsc.VectorSubcoreMesh(core_axis_name="c", subcore_axis_name="s")

    zr = 48
    assert rpt % zr == 0 and rem <= zr

    @functools.partial(
        pl.kernel,
        out_type=jax.ShapeDtypeStruct((_NC, n, d), jnp.float32),
        mesh=mesh,
        scratch_types=[
            pltpu.VMEM((zr, d), jnp.float32),
            pltpu.VMEM((_C, d), jnp.float32),
            pltpu.VMEM((_C,), jnp.int32),
            pltpu.VMEM((_C,), jnp.int32),
            pltpu.VMEM_SHARED((n, d), jnp.float32),
            pltpu.SemaphoreType.DMA,
        ],
    )
    def k(x_hbm, src_hbm, dst_hbm, out_hbm, zbuf, rows_v, sidx, didx, acc, sem):
        cid = lax.axis_index("c")
        sid = lax.axis_index("s")

        zero16 = jnp.zeros((16,), jnp.float32)

        def zrow(r, _):
            for cc in range(d // 16):
                zbuf[r, pl.ds(cc * 16, 16)] = zero16
            return 0

        lax.fori_loop(0, zr, zrow, 0)

        def zcopy(t, _):
            pltpu.sync_copy(zbuf, acc.at[pl.ds(sid * rpt + t * zr, zr)])
            return 0

        lax.fori_loop(0, rpt // zr, zcopy, 0)

        @pl.when(sid == 0)
        def _():
            pltpu.sync_copy(zbuf.at[pl.ds(0, rem)], acc.at[pl.ds(_NS * rpt, rem)])

        plsc.subcore_barrier()

        def body(t, _):
            rel = sid + _NS * t

            @pl.when(rel < chunks_per_core)
            def _():
                off = (cid * chunks_per_core + rel) * _C
                pltpu.sync_copy(src_hbm.at[pl.ds(off, _C)], sidx)
                pltpu.sync_copy(dst_hbm.at[pl.ds(off, _C)], didx)
                pltpu.async_copy(x_hbm.at[sidx], rows_v, sem).wait()
                pltpu.sync_copy(rows_v, acc.at[didx], add=True)

            return 0

        lax.fori_loop(0, tmax, body, 0)
        plsc.subcore_barrier()

        pltpu.sync_copy(
            acc.at[pl.ds(sid * rpt, rpt)], out_hbm.at[cid, pl.ds(sid * rpt, rpt)]
        )

        @pl.when(sid == 0)
        def _():
            pltpu.sync_copy(
                acc.at[pl.ds(_NS * rpt, rem)], out_hbm.at[cid, pl.ds(_NS * rpt, rem)]
            )

    return k(x, src, dst)


def _tc_layer(p, xin, wlT, wrT, bl, scale, shift, relu):
    n, d = xin.shape
    br = 1000
    assert n % br == 0

    def body(p_ref, x_ref, wl_ref, wr_ref, bl_ref, sc_ref, sh_ref, o_ref):
        agg = p_ref[0] + p_ref[1]
        y = jnp.dot(agg, wl_ref[...], preferred_element_type=jnp.float32)
        y = y + jnp.dot(x_ref[...], wr_ref[...], preferred_element_type=jnp.float32)
        y = y + bl_ref[...]
        nrm = jnp.sqrt(jnp.sum(y * y, axis=1, keepdims=True))
        y = y / jnp.maximum(nrm, 1e-12)
        y = y * sc_ref[...] + sh_ref[...]
        if relu:
            y = jnp.maximum(y, 0.0)
        o_ref[...] = y

    return pl.pallas_call(
        body,
        grid=(n // br,),
        in_specs=[
            pl.BlockSpec((2, br, d), lambda i: (0, i, 0)),
            pl.BlockSpec((br, d), lambda i: (i, 0)),
            pl.BlockSpec((d, d), lambda i: (0, 0)),
            pl.BlockSpec((d, d), lambda i: (0, 0)),
            pl.BlockSpec((1, d), lambda i: (0, 0)),
            pl.BlockSpec((1, d), lambda i: (0, 0)),
            pl.BlockSpec((1, d), lambda i: (0, 0)),
        ],
        out_specs=pl.BlockSpec((br, d), lambda i: (i, 0)),
        out_shape=jax.ShapeDtypeStruct((n, d), jnp.float32),
    )(p, xin, wlT, wrT, bl, scale, shift)


def kernel(x, edge_index, Wl1, bl1, Wr1, g1, be1, m1, v1, Wl2, bl2, Wr2, g2, be2, m2, v2):
    n, d = x.shape
    src = edge_index[0]
    dst = edge_index[1]
    scale1 = g1 / jnp.sqrt(v1 + 1e-5)
    shift1 = be1 - m1 * scale1
    scale2 = g2 / jnp.sqrt(v2 + 1e-5)
    shift2 = be2 - m2 * scale2

    p1 = _sc_scatter(x, src, dst)
    h = _tc_layer(p1, x, Wl1.T, Wr1.T, bl1[None], scale1[None], shift1[None], True)
    p2 = _sc_scatter(h, src, dst)
    return _tc_layer(p2, h, Wl2.T, Wr2.T, bl2[None], scale2[None], shift2[None], False)
